# baseline (speedup 1.0000x reference)
import jax
import jax.numpy as jnp
from jax import lax
from jax.experimental import pallas as pl
from jax.experimental.pallas import tpu as pltpu
from jax.experimental.pallas import tpu_sc as plsc

N = 10000
E = 320000
D = 128

NC = 2
NS = 16
NW = NC * NS

CH = 128
CPT = 80
EPT = CPT * CH
EPAD = NW * EPT

NP = 10112
RPT = NP // NS
_RCHUNKS = ((0, 128), (128, 128), (256, 128), (384, 128), (512, 120))

RB = 1264


def _zero_acc(zrows, rows, acc_sh, sid):
    pltpu.sync_copy(zrows, rows)
    for o_, s_ in _RCHUNKS:
        r0 = sid * RPT + o_
        pltpu.sync_copy(rows.at[pl.ds(0, s_)], acc_sh.at[pl.ds(r0, s_)])


def _copy_out(out_acc, rows, acc_sh, cid, sid):
    for o_, s_ in _RCHUNKS:
        r0 = sid * RPT + o_
        pltpu.sync_copy(acc_sh.at[pl.ds(r0, s_)], rows.at[pl.ds(0, s_)])
        pltpu.sync_copy(rows.at[pl.ds(0, s_)],
                        out_acc.at[pl.ds(cid * NP + r0, s_)])


NB = 2
BLK = 40
CPT_A = 40
CPT_B = 120


def _sc_agg_body(h, edges, zrows, out_acc,
                 eall, r0, r1, acc_sh, sg0, sg1, ss0, ss1):
    rows = (r0, r1)
    sg = (sg0, sg1)
    ss = (ss0, ss1)
    cid = lax.axis_index("c")
    sid = lax.axis_index("s")

    _zero_acc(zrows, r0, acc_sh, sid)
    plsc.subcore_barrier()

    def g_start(c, b):
        pltpu.async_copy(h.at[eall.at[2 * c]], rows[b], sg[b])

    def g_wait(c, b):
        pltpu.make_async_copy(h.at[eall.at[2 * c]], rows[b], sg[b]).wait()

    def s_start(c, b):
        pltpu.async_copy(rows[b], acc_sh.at[eall.at[2 * c + 1]], ss[b],
                         add=True)

    def s_wait(c, b):
        pltpu.make_async_copy(rows[b], acc_sh.at[eall.at[2 * c + 1]],
                              ss[b]).wait()

    def run_block(row0):
        pltpu.sync_copy(edges.at[pl.ds(pl.multiple_of(row0, 16), 2 * BLK)],
                        eall)
        for b in range(NB):
            g_start(b, b)

        def body(i, carry):
            c0 = NB * i
            for b in range(NB):
                g_wait(c0 + b, b)
                s_start(c0 + b, b)
            for b in range(NB):
                s_wait(c0 + b, b)
                g_start(c0 + NB + b, b)
            return carry

        lax.fori_loop(0, BLK // NB - 1, body, 0)
        c0 = BLK - NB
        for b in range(NB):
            g_wait(c0 + b, b)
            s_start(c0 + b, b)
        for b in range(NB):
            s_wait(c0 + b, b)

    @pl.when(cid == 0)
    def _():
        for blk in range(CPT_A // BLK):
            run_block(sid * (2 * CPT_A) + blk * 2 * BLK)

    @pl.when(cid == 1)
    def _():
        for blk in range(CPT_B // BLK):
            run_block(NS * 2 * CPT_A + sid * (2 * CPT_B) + blk * 2 * BLK)

    plsc.subcore_barrier()
    _copy_out(out_acc, r0, acc_sh, cid, sid)


_sc_agg = pl.kernel(
    _sc_agg_body,
    out_type=[jax.ShapeDtypeStruct((NC * NP, D), jnp.float32)],
    mesh=plsc.VectorSubcoreMesh(core_axis_name="c", subcore_axis_name="s"),
    scratch_types=[
        pltpu.VMEM((2 * BLK, CH), jnp.int32),
        pltpu.VMEM((CH, D), jnp.float32),
        pltpu.VMEM((CH, D), jnp.float32),
        pltpu.VMEM_SHARED((NP, D), jnp.float32),
        pltpu.SemaphoreType.DMA,
        pltpu.SemaphoreType.DMA,
        pltpu.SemaphoreType.DMA,
        pltpu.SemaphoreType.DMA,
    ],
)


def _sc_deg_body(edges, zrows, ones_hbm, out_acc,
                 eall, onesv, acc_sh, ssem):
    cid = lax.axis_index("c")
    sid = lax.axis_index("s")
    wid = cid * NS + sid

    _zero_acc(zrows, onesv, acc_sh, sid)
    pltpu.sync_copy(ones_hbm, onesv)
    plsc.subcore_barrier()

    pltpu.sync_copy(edges.at[pl.ds(pl.multiple_of(wid * 2 * CPT, 16), 2 * CPT)],
                    eall)

    def body(i, carry):
        for j in range(8):
            pltpu.async_copy(onesv, acc_sh.at[eall.at[2 * (8 * i + j) + 1]],
                             ssem, add=True)
        for j in range(8):
            pltpu.make_async_copy(
                onesv, acc_sh.at[eall.at[2 * (8 * i + j) + 1]],
                ssem).wait()
        return carry

    lax.fori_loop(0, CPT // 8, body, 0)

    plsc.subcore_barrier()
    _copy_out(out_acc, onesv, acc_sh, cid, sid)


_sc_deg = pl.kernel(
    _sc_deg_body,
    out_type=[jax.ShapeDtypeStruct((NC * NP, D), jnp.float32)],
    mesh=plsc.VectorSubcoreMesh(core_axis_name="c", subcore_axis_name="s"),
    scratch_types=[
        pltpu.VMEM((2 * CPT, CH), jnp.int32),
        pltpu.VMEM((CH, D), jnp.float32),
        pltpu.VMEM_SHARED((NP, D), jnp.float32),
        pltpu.SemaphoreType.DMA,
    ],
)


def _combine(acc, degp, h, Wl, Wr, b, relu):
    def body(p0r, p1r, d0r, d1r, hr, wlr, wrr, br, o):
        deg = d0r[:, :1] + d1r[:, :1]
        inv = 1.0 / jnp.maximum(deg, 1.0)
        mean = (p0r[...] + p1r[...]) * inv
        out = jnp.dot(mean, wlr[...], preferred_element_type=jnp.float32)
        out = out + jnp.dot(hr[...], wrr[...], preferred_element_type=jnp.float32)
        out = out + br[...]
        if relu:
            out = jnp.maximum(out, 0.0)
        o[...] = out

    p0, p1 = acc[:NP], acc[NP:]
    d0, d1 = degp[:NP], degp[NP:]
    return pl.pallas_call(
        body,
        grid=(NP // RB,),
        in_specs=[
            pl.BlockSpec((RB, D), lambda i: (i, 0)),
            pl.BlockSpec((RB, D), lambda i: (i, 0)),
            pl.BlockSpec((RB, D), lambda i: (i, 0)),
            pl.BlockSpec((RB, D), lambda i: (i, 0)),
            pl.BlockSpec((RB, D), lambda i: (i, 0)),
            pl.BlockSpec((D, D), lambda i: (0, 0)),
            pl.BlockSpec((D, D), lambda i: (0, 0)),
            pl.BlockSpec((1, D), lambda i: (0, 0)),
        ],
        out_specs=pl.BlockSpec((RB, D), lambda i: (i, 0)),
        out_shape=jax.ShapeDtypeStruct((NP, D), jnp.float32),
    )(p0, p1, d0, d1, h, Wl, Wr, b.reshape(1, D))


def kernel(x, edge_index, Wl0, Wr0, b0, Wl1, Wr1, b1, Wl2, Wr2, b2):
    src = edge_index[0].astype(jnp.int32)
    dst = edge_index[1].astype(jnp.int32)
    pad = EPAD - E
    srcp = jnp.concatenate([src, jnp.zeros((pad,), jnp.int32)])
    pad_dst = N + (jnp.arange(pad, dtype=jnp.int32) % (NP - N))
    dstp = jnp.concatenate([dst, pad_dst])
    edges = jnp.stack(
        [srcp.reshape(-1, CH), dstp.reshape(-1, CH)], axis=1
    ).reshape(-1, CH)
    hp = jnp.concatenate([x, jnp.zeros((NP - N, D), jnp.float32)], axis=0)

    zrows = jnp.zeros((CH, D), jnp.float32)
    ones = jnp.ones((CH, D), jnp.float32)

    (degp,) = _sc_deg(edges, zrows, ones)
    (acc0,) = _sc_agg(hp, edges, zrows)
    h1 = _combine(acc0, degp, hp, Wl0, Wr0, b0, relu=True)
    (acc1,) = _sc_agg(h1, edges, zrows)
    h2 = _combine(acc1, degp, h1, Wl1, Wr1, b1, relu=True)
    (acc2,) = _sc_agg(h2, edges, zrows)
    h3 = _combine(acc2, degp, h2, Wl2, Wr2, b2, relu=False)
    return h3[:N]


# speedup vs baseline: 1.1801x; 1.1801x over previous
import jax
import jax.numpy as jnp
from jax import lax
from jax.experimental import pallas as pl
from jax.experimental.pallas import tpu as pltpu
from jax.experimental.pallas import tpu_sc as plsc

N = 10000
E = 320000
D = 128

NC = 2
NS = 16
NW = NC * NS

CH = 128
CPT = 80
EPT = CPT * CH
EPAD = NW * EPT

NP = 10112
RPT = NP // NS
_RCHUNKS = ((0, 128), (128, 128), (256, 128), (384, 128), (512, 120))

RB = 1264


def _zero_acc(zrows, rows, acc_sh, sid):
    pltpu.sync_copy(zrows, rows)
    for o_, s_ in _RCHUNKS:
        r0 = sid * RPT + o_
        pltpu.sync_copy(rows.at[pl.ds(0, s_)], acc_sh.at[pl.ds(r0, s_)])


def _copy_out(out_acc, rows, acc_sh, cid, sid):
    for o_, s_ in _RCHUNKS:
        r0 = sid * RPT + o_
        pltpu.sync_copy(acc_sh.at[pl.ds(r0, s_)], rows.at[pl.ds(0, s_)])
        pltpu.sync_copy(rows.at[pl.ds(0, s_)],
                        out_acc.at[pl.ds(cid * NP + r0, s_)])


NB = 2
BLK = 40
CPT_A = 120
CPT_B = 40


def _sc_agg_body(h, edges, zrows, out_acc,
                 eall, r0, r1, acc_sh, sg0, sg1, ss0, ss1):
    rows = (r0, r1)
    sg = (sg0, sg1)
    ss = (ss0, ss1)
    cid = lax.axis_index("c")
    sid = lax.axis_index("s")

    _zero_acc(zrows, r0, acc_sh, sid)
    plsc.subcore_barrier()

    def g_start(c, b):
        pltpu.async_copy(h.at[eall.at[2 * c]], rows[b], sg[b])

    def g_wait(c, b):
        pltpu.make_async_copy(h.at[eall.at[2 * c]], rows[b], sg[b]).wait()

    def s_start(c, b):
        pltpu.async_copy(rows[b], acc_sh.at[eall.at[2 * c + 1]], ss[b],
                         add=True)

    def s_wait(c, b):
        pltpu.make_async_copy(rows[b], acc_sh.at[eall.at[2 * c + 1]],
                              ss[b]).wait()

    def run_block(row0):
        pltpu.sync_copy(edges.at[pl.ds(pl.multiple_of(row0, 16), 2 * BLK)],
                        eall)
        for b in range(NB):
            g_start(b, b)

        def body(i, carry):
            c0 = NB * i
            for b in range(NB):
                g_wait(c0 + b, b)
                s_start(c0 + b, b)
            for b in range(NB):
                s_wait(c0 + b, b)
                g_start(c0 + NB + b, b)
            return carry

        lax.fori_loop(0, BLK // NB - 1, body, 0)
        c0 = BLK - NB
        for b in range(NB):
            g_wait(c0 + b, b)
            s_start(c0 + b, b)
        for b in range(NB):
            s_wait(c0 + b, b)

    @pl.when(cid == 0)
    def _():
        for blk in range(CPT_A // BLK):
            run_block(sid * (2 * CPT_A) + blk * 2 * BLK)

    @pl.when(cid == 1)
    def _():
        for blk in range(CPT_B // BLK):
            run_block(NS * 2 * CPT_A + sid * (2 * CPT_B) + blk * 2 * BLK)

    plsc.subcore_barrier()
    _copy_out(out_acc, r0, acc_sh, cid, sid)


_sc_agg = pl.kernel(
    _sc_agg_body,
    out_type=[jax.ShapeDtypeStruct((NC * NP, D), jnp.float32)],
    mesh=plsc.VectorSubcoreMesh(core_axis_name="c", subcore_axis_name="s"),
    scratch_types=[
        pltpu.VMEM((2 * BLK, CH), jnp.int32),
        pltpu.VMEM((CH, D), jnp.float32),
        pltpu.VMEM((CH, D), jnp.float32),
        pltpu.VMEM_SHARED((NP, D), jnp.float32),
        pltpu.SemaphoreType.DMA,
        pltpu.SemaphoreType.DMA,
        pltpu.SemaphoreType.DMA,
        pltpu.SemaphoreType.DMA,
    ],
)


def _sc_deg_body(edges, zrows, ones_hbm, out_acc,
                 eall, onesv, acc_sh, ssem):
    cid = lax.axis_index("c")
    sid = lax.axis_index("s")
    wid = cid * NS + sid

    _zero_acc(zrows, onesv, acc_sh, sid)
    pltpu.sync_copy(ones_hbm, onesv)
    plsc.subcore_barrier()

    pltpu.sync_copy(edges.at[pl.ds(pl.multiple_of(wid * 2 * CPT, 16), 2 * CPT)],
                    eall)

    def body(i, carry):
        for j in range(8):
            pltpu.async_copy(onesv, acc_sh.at[eall.at[2 * (8 * i + j) + 1]],
                             ssem, add=True)
        for j in range(8):
            pltpu.make_async_copy(
                onesv, acc_sh.at[eall.at[2 * (8 * i + j) + 1]],
                ssem).wait()
        return carry

    lax.fori_loop(0, CPT // 8, body, 0)

    plsc.subcore_barrier()
    _copy_out(out_acc, onesv, acc_sh, cid, sid)


_sc_deg = pl.kernel(
    _sc_deg_body,
    out_type=[jax.ShapeDtypeStruct((NC * NP, D), jnp.float32)],
    mesh=plsc.VectorSubcoreMesh(core_axis_name="c", subcore_axis_name="s"),
    scratch_types=[
        pltpu.VMEM((2 * CPT, CH), jnp.int32),
        pltpu.VMEM((CH, D), jnp.float32),
        pltpu.VMEM_SHARED((NP, D), jnp.float32),
        pltpu.SemaphoreType.DMA,
    ],
)


def _combine(acc, degp, h, Wl, Wr, b, relu):
    def body(p0r, p1r, d0r, d1r, hr, wlr, wrr, br, o):
        deg = d0r[:, :1] + d1r[:, :1]
        inv = 1.0 / jnp.maximum(deg, 1.0)
        mean = (p0r[...] + p1r[...]) * inv
        out = jnp.dot(mean, wlr[...], preferred_element_type=jnp.float32)
        out = out + jnp.dot(hr[...], wrr[...], preferred_element_type=jnp.float32)
        out = out + br[...]
        if relu:
            out = jnp.maximum(out, 0.0)
        o[...] = out

    p0, p1 = acc[:NP], acc[NP:]
    d0, d1 = degp[:NP], degp[NP:]
    return pl.pallas_call(
        body,
        grid=(NP // RB,),
        in_specs=[
            pl.BlockSpec((RB, D), lambda i: (i, 0)),
            pl.BlockSpec((RB, D), lambda i: (i, 0)),
            pl.BlockSpec((RB, D), lambda i: (i, 0)),
            pl.BlockSpec((RB, D), lambda i: (i, 0)),
            pl.BlockSpec((RB, D), lambda i: (i, 0)),
            pl.BlockSpec((D, D), lambda i: (0, 0)),
            pl.BlockSpec((D, D), lambda i: (0, 0)),
            pl.BlockSpec((1, D), lambda i: (0, 0)),
        ],
        out_specs=pl.BlockSpec((RB, D), lambda i: (i, 0)),
        out_shape=jax.ShapeDtypeStruct((NP, D), jnp.float32),
    )(p0, p1, d0, d1, h, Wl, Wr, b.reshape(1, D))


def kernel(x, edge_index, Wl0, Wr0, b0, Wl1, Wr1, b1, Wl2, Wr2, b2):
    src = edge_index[0].astype(jnp.int32)
    dst = edge_index[1].astype(jnp.int32)
    pad = EPAD - E
    srcp = jnp.concatenate([src, jnp.zeros((pad,), jnp.int32)])
    pad_dst = N + (jnp.arange(pad, dtype=jnp.int32) % (NP - N))
    dstp = jnp.concatenate([dst, pad_dst])
    edges = jnp.stack(
        [srcp.reshape(-1, CH), dstp.reshape(-1, CH)], axis=1
    ).reshape(-1, CH)
    hp = jnp.concatenate([x, jnp.zeros((NP - N, D), jnp.float32)], axis=0)

    zrows = jnp.zeros((CH, D), jnp.float32)
    ones = jnp.ones((CH, D), jnp.float32)

    (degp,) = _sc_deg(edges, zrows, ones)
    (acc0,) = _sc_agg(hp, edges, zrows)
    h1 = _combine(acc0, degp, hp, Wl0, Wr0, b0, relu=True)
    (acc1,) = _sc_agg(h1, edges, zrows)
    h2 = _combine(acc1, degp, h1, Wl1, Wr1, b1, relu=True)
    (acc2,) = _sc_agg(h2, edges, zrows)
    h3 = _combine(acc2, degp, h2, Wl2, Wr2, b2, relu=False)
    return h3[:N]
